# Initial kernel scaffold; baseline (speedup 1.0000x reference)
import jax
import jax.numpy as jnp
from jax.experimental import pallas as pl


def kernel(x, sinusoid):
    raise NotImplementedError("write your pallas kernel here")


# speedup vs baseline: 6.1252x; 6.1252x over previous
import functools

import jax
import jax.numpy as jnp
from jax import lax
from jax.experimental import pallas as pl
from jax.experimental.pallas import tpu as pltpu
from jax.experimental.pallas import tpu_sc as plsc

D = 32
IDX_W = 128
B_ROWS = 25600
NW = 32
ROWS_PER_W = B_ROWS // NW
CR = 8
N_CHUNKS = ROWS_PER_W // CR


def _sc_gather(idx2d, table):
    mesh = plsc.VectorSubcoreMesh(core_axis_name="c", subcore_axis_name="s")

    @functools.partial(
        pl.kernel,
        mesh=mesh,
        compiler_params=pltpu.CompilerParams(use_tc_tiling_on_sc=False),
        out_type=jax.ShapeDtypeStruct((B_ROWS, IDX_W, D), jnp.float32),
        scratch_types=[
            pltpu.VMEM((CR, IDX_W), jnp.int32),
            pltpu.VMEM((CR, IDX_W, D), jnp.float32),
            pltpu.SemaphoreType.DMA,
        ],
    )
    def k(idx_hbm, table_hbm, out_hbm, idx_v, rows_v, sem):
        wid = lax.axis_index("s") * 2 + lax.axis_index("c")
        base = wid * ROWS_PER_W

        def chunk(g, carry):
            row0 = base + g * CR
            pltpu.sync_copy(idx_hbm.at[pl.ds(row0, CR)], idx_v)
            copies = [
                pltpu.async_copy(table_hbm.at[idx_v.at[j]], rows_v.at[j], sem)
                for j in range(CR)
            ]
            for c in copies:
                c.wait()
            pltpu.sync_copy(rows_v, out_hbm.at[pl.ds(row0, CR)])
            return carry

        lax.fori_loop(0, N_CHUNKS, chunk, 0)

    return k(idx2d, table)


def kernel(x, sinusoid):
    idx2d = x.reshape(B_ROWS, IDX_W).astype(jnp.int32)
    out = _sc_gather(idx2d, sinusoid)
    return out.reshape(x.shape[0], x.shape[1], D)


# speedup vs baseline: 6.3459x; 1.0360x over previous
import functools

import jax
import jax.numpy as jnp
from jax import lax
from jax.experimental import pallas as pl
from jax.experimental.pallas import tpu as pltpu
from jax.experimental.pallas import tpu_sc as plsc

D = 32
IDX_W = 128
B_ROWS = 25600
NW = 32
ROWS_PER_W = B_ROWS // NW
CR = 8
N_CHUNKS = ROWS_PER_W // CR


def _sc_gather(idx2d, table):
    mesh = plsc.VectorSubcoreMesh(core_axis_name="c", subcore_axis_name="s")

    @functools.partial(
        pl.kernel,
        mesh=mesh,
        compiler_params=pltpu.CompilerParams(use_tc_tiling_on_sc=False),
        out_type=jax.ShapeDtypeStruct((B_ROWS, IDX_W, D), jnp.float32),
        scratch_types=[
            pltpu.VMEM((CR, IDX_W), jnp.int32),
            pltpu.VMEM((CR, IDX_W), jnp.int32),
            pltpu.VMEM((CR, IDX_W, D), jnp.float32),
            pltpu.VMEM((CR, IDX_W, D), jnp.float32),
            pltpu.SemaphoreType.DMA,
            pltpu.SemaphoreType.DMA,
            pltpu.SemaphoreType.DMA,
            pltpu.SemaphoreType.DMA,
        ],
    )
    def k(idx_hbm, table_hbm, out_hbm, idx0, idx1, rows0, rows1,
          sg0, sg1, so0, so1):
        wid = lax.axis_index("s") * 2 + lax.axis_index("c")
        base = wid * ROWS_PER_W
        idx_v = (idx0, idx1)
        rows_v = (rows0, rows1)
        sg = (sg0, sg1)
        so = (so0, so1)

        def fire(b, row0):
            pltpu.sync_copy(idx_hbm.at[pl.ds(row0, CR)], idx_v[b])
            for j in range(CR):
                pltpu.async_copy(table_hbm.at[idx_v[b].at[j]],
                                 rows_v[b].at[j], sg[b])

        def drain_gather(b):
            for j in range(CR):
                pltpu.make_async_copy(table_hbm.at[idx_v[b].at[j]],
                                      rows_v[b].at[j], sg[b]).wait()

        def drain_store(b):
            pltpu.make_async_copy(out_hbm.at[pl.ds(0, CR)],
                                  rows_v[b], so[b]).wait()

        fire(0, base)

        def outer(g2, carry):
            for b in range(2):
                g = g2 * 2 + b
                nb = 1 - b

                @pl.when(g + 1 < N_CHUNKS)
                def _prep():
                    @pl.when(g + 1 >= 2)
                    def _reuse():
                        drain_store(nb)
                    fire(nb, base + (g + 1) * CR)

                drain_gather(b)
                pltpu.async_copy(rows_v[b], out_hbm.at[pl.ds(base + g * CR, CR)],
                                 so[b])
            return carry

        lax.fori_loop(0, N_CHUNKS // 2, outer, 0)
        drain_store(0)
        drain_store(1)

    return k(idx2d, table)


def kernel(x, sinusoid):
    idx2d = x.reshape(B_ROWS, IDX_W).astype(jnp.int32)
    out = _sc_gather(idx2d, sinusoid)
    return out.reshape(x.shape[0], x.shape[1], D)
